# baseline (speedup 1.0000x reference)
import functools

import jax
import jax.numpy as jnp
from jax import lax
from jax.experimental import pallas as pl
from jax.experimental.pallas import tpu as pltpu
from jax.experimental.pallas import tpu_sc as plsc

NC = 2
NS = 16
NW = NC * NS
K = 128
C = 8
BLK = 2000


def _sc_mesh():
    return plsc.VectorSubcoreMesh(
        core_axis_name="c", subcore_axis_name="s", num_cores=NC, num_subcores=NS
    )


def _make_prop_kernel(npad, d, nwin):
    rpt = npad // NS
    nchunk = nwin // C

    @functools.partial(
        pl.kernel,
        out_type=jax.ShapeDtypeStruct((NC, npad, d), jnp.float32),
        mesh=_sc_mesh(),
        scratch_types=[
            pltpu.VMEM((nwin, K), jnp.int32),
            pltpu.VMEM((nwin, K), jnp.int32),
            pltpu.VMEM((K, d), jnp.float32),
            pltpu.VMEM_SHARED((npad, d), jnp.float32),
            pltpu.SemaphoreType.DMA,
        ],
    )
    def prop_kernel(
        g_hbm, src_hbm, dst_hbm, zeros_hbm, out_hbm,
        src_v, dst_v, rows_v, acc_sh, sem,
    ):
        c = lax.axis_index("c")
        s = lax.axis_index("s")
        wid = s * NC + c
        pltpu.sync_copy(src_hbm.at[wid], src_v)
        pltpu.sync_copy(dst_hbm.at[wid], dst_v)
        pltpu.sync_copy(
            zeros_hbm.at[pl.ds(s * rpt, rpt)], acc_sh.at[pl.ds(s * rpt, rpt)]
        )
        plsc.subcore_barrier()

        @pl.loop(0, nwin)
        def _(w):
            pltpu.async_copy(g_hbm.at[src_v.at[w]], rows_v, sem).wait()
            pltpu.sync_copy(rows_v, acc_sh.at[dst_v.at[w]], add=True)

        plsc.subcore_barrier()
        pltpu.sync_copy(
            acc_sh.at[pl.ds(s * rpt, rpt)], out_hbm.at[c, pl.ds(s * rpt, rpt)]
        )

    return prop_kernel


def _dinv_from_deg(deg_ref):
    deg = deg_ref[0, :, 0:1] + deg_ref[1, :, 0:1] + 1.0
    return lax.rsqrt(deg)


def _deg_spec():
    return pl.BlockSpec((2, BLK, 128), lambda i: (0, i, 0))


def _tc1_body(deg_ref, x_ref, w_ref, h_ref, g_ref):
    dinv = _dinv_from_deg(deg_ref)
    h = jnp.dot(x_ref[...], w_ref[...], preferred_element_type=jnp.float32)
    h_ref[...] = h
    g_ref[...] = h * dinv


def _tc2_body(deg_ref, s_ref, h_ref, w_ref, b_ref, h2_ref, g2_ref):
    dinv = _dinv_from_deg(deg_ref)
    stot = s_ref[0] + s_ref[1]
    out1 = jnp.maximum(stot * dinv + h_ref[...] * (dinv * dinv) + b_ref[...], 0.0)
    h2 = jnp.dot(out1, w_ref[...], preferred_element_type=jnp.float32)
    h2_ref[...] = h2
    g2_ref[...] = h2 * dinv


def _tc3_body(deg_ref, s_ref, h_ref, b_ref, o_ref):
    dinv = _dinv_from_deg(deg_ref)
    stot = s_ref[0] + s_ref[1]
    o_ref[...] = jnp.maximum(
        stot * dinv + h_ref[...] * (dinv * dinv) + b_ref[...], 0.0
    )


def _row_spec(d):
    return pl.BlockSpec((BLK, d), lambda i: (i, 0))


def _pair_spec(d):
    return pl.BlockSpec((2, BLK, d), lambda i: (0, i, 0))


def _full_spec(r, c):
    return pl.BlockSpec((r, c), lambda i: (0, 0))


def kernel(x, edge_index, W1, b1, W2, b2):
    n, d = x.shape
    e = edge_index.shape[1]
    grid = (n // BLK,)
    npad = ((n // NS + 7) // 8 * 8) * NS

    epw = e // NW
    nwin_min = (epw + K - 1) // K
    nwin = (nwin_min + 2 * C - 1) // (2 * C) * (2 * C)
    padlen = nwin * K - epw
    srcw = edge_index[0].reshape(NW, epw)
    dstw = edge_index[1].reshape(NW, epw)
    src2 = jnp.concatenate(
        [srcw, jnp.zeros((NW, padlen), jnp.int32)], axis=1
    ).reshape(NW, nwin, K)
    dst2 = jnp.concatenate(
        [dstw, jnp.full((NW, padlen), npad - 1, jnp.int32)], axis=1
    ).reshape(NW, nwin, K)
    zeros_d = jnp.zeros((npad, d), jnp.float32)
    ones_nd = jnp.ones((n, d), jnp.float32)
    b1r = b1.reshape(1, d)
    b2r = b2.reshape(1, d)

    prop = _make_prop_kernel(npad, d, nwin)
    deg16 = prop(ones_nd, src2, dst2, zeros_d)

    h1, g1 = pl.pallas_call(
        _tc1_body,
        grid=grid,
        in_specs=[_deg_spec(), _row_spec(d), _full_spec(d, d)],
        out_specs=[_row_spec(d), _row_spec(d)],
        out_shape=[
            jax.ShapeDtypeStruct((n, d), jnp.float32),
            jax.ShapeDtypeStruct((n, d), jnp.float32),
        ],
    )(deg16, x, W1)

    s1 = prop(g1, src2, dst2, zeros_d)

    h2, g2 = pl.pallas_call(
        _tc2_body,
        grid=grid,
        in_specs=[
            _deg_spec(),
            _pair_spec(d),
            _row_spec(d),
            _full_spec(d, d),
            _full_spec(1, d),
        ],
        out_specs=[_row_spec(d), _row_spec(d)],
        out_shape=[
            jax.ShapeDtypeStruct((n, d), jnp.float32),
            jax.ShapeDtypeStruct((n, d), jnp.float32),
        ],
    )(deg16, s1, h1, W2, b1r)

    s2 = prop(g2, src2, dst2, zeros_d)

    out = pl.pallas_call(
        _tc3_body,
        grid=grid,
        in_specs=[_deg_spec(), _pair_spec(d), _row_spec(d), _full_spec(1, d)],
        out_specs=_row_spec(d),
        out_shape=jax.ShapeDtypeStruct((n, d), jnp.float32),
    )(deg16, s2, h2, b2r)

    return out


# speedup vs baseline: 2.3838x; 2.3838x over previous
import functools

import jax
import jax.numpy as jnp
from jax import lax
from jax.experimental import pallas as pl
from jax.experimental.pallas import tpu as pltpu
from jax.experimental.pallas import tpu_sc as plsc

NC = 2
NS = 16
NW = NC * NS
K = 128
C = 8
BLK = 2000


def _sc_mesh():
    return plsc.VectorSubcoreMesh(
        core_axis_name="c", subcore_axis_name="s", num_cores=NC, num_subcores=NS
    )


def _make_prop_kernel(npad, d, nwin):
    rpt = npad // NS
    nchunk = nwin // C

    @functools.partial(
        pl.kernel,
        out_type=jax.ShapeDtypeStruct((NC, npad, d), jnp.float32),
        mesh=_sc_mesh(),
        scratch_types=[
            pltpu.VMEM((nwin, K), jnp.int32),
            pltpu.VMEM((nwin, K), jnp.int32),
            pltpu.VMEM((K, d), jnp.float32),
            pltpu.VMEM_SHARED((npad, d), jnp.float32),
            pltpu.SemaphoreType.DMA,
        ],
    )
    def prop_kernel(
        g_hbm, src_hbm, dst_hbm, zeros_hbm, out_hbm,
        src_v, dst_v, rows_v, acc_sh, sem,
    ):
        c = lax.axis_index("c")
        s = lax.axis_index("s")
        wid = s * NC + c
        pltpu.sync_copy(src_hbm.at[wid], src_v)
        pltpu.sync_copy(dst_hbm.at[wid], dst_v)
        pltpu.sync_copy(
            zeros_hbm.at[pl.ds(s * rpt, rpt)], acc_sh.at[pl.ds(s * rpt, rpt)]
        )
        plsc.subcore_barrier()

        @pl.loop(0, nwin)
        def _(w):
            pltpu.async_copy(g_hbm.at[src_v.at[w]], rows_v, sem).wait()
            pltpu.sync_copy(rows_v, acc_sh.at[dst_v.at[w]], add=True)

        plsc.subcore_barrier()
        pltpu.sync_copy(
            acc_sh.at[pl.ds(s * rpt, rpt)], out_hbm.at[c, pl.ds(s * rpt, rpt)]
        )

    return prop_kernel


def _dinv_from_deg(deg_ref):
    deg = deg_ref[0, :, 0:1] + deg_ref[1, :, 0:1] + 1.0
    return lax.rsqrt(deg)


def _deg_spec():
    return pl.BlockSpec((2, BLK, 128), lambda i: (0, i, 0))


def _tc1_body(deg_ref, x_ref, w_ref, h_ref, g_ref):
    dinv = _dinv_from_deg(deg_ref)
    h = jnp.dot(x_ref[...], w_ref[...], preferred_element_type=jnp.float32)
    h_ref[...] = h
    g_ref[...] = h * dinv


def _tc2_body(deg_ref, s_ref, h_ref, w_ref, b_ref, h2_ref, g2_ref):
    dinv = _dinv_from_deg(deg_ref)
    stot = s_ref[0] + s_ref[1]
    out1 = jnp.maximum(stot * dinv + h_ref[...] * (dinv * dinv) + b_ref[...], 0.0)
    h2 = jnp.dot(out1, w_ref[...], preferred_element_type=jnp.float32)
    h2_ref[...] = h2
    g2_ref[...] = h2 * dinv


def _tc3_body(deg_ref, s_ref, h_ref, b_ref, o_ref):
    dinv = _dinv_from_deg(deg_ref)
    stot = s_ref[0] + s_ref[1]
    o_ref[...] = jnp.maximum(
        stot * dinv + h_ref[...] * (dinv * dinv) + b_ref[...], 0.0
    )


def _row_spec(d):
    return pl.BlockSpec((BLK, d), lambda i: (i, 0))


def _pair_spec(d):
    return pl.BlockSpec((2, BLK, d), lambda i: (0, i, 0))


def _full_spec(r, c):
    return pl.BlockSpec((r, c), lambda i: (0, 0))


def kernel(x, edge_index, W1, b1, W2, b2):
    n, d = x.shape
    e = edge_index.shape[1]
    grid = (n // BLK,)
    npad = ((n // NS + 7) // 8 * 8) * NS

    epw = e // NW
    nwin_min = (epw + K - 1) // K
    nwin = (nwin_min + 2 * C - 1) // (2 * C) * (2 * C)
    padlen = nwin * K - epw
    srcw = edge_index[0].reshape(NW, epw)
    dstw = edge_index[1].reshape(NW, epw)
    pad_src = jnp.broadcast_to(
        jnp.arange(padlen, dtype=jnp.int32)[None, :] % n, (NW, padlen)
    )
    pad_dst = jnp.broadcast_to(
        n + jnp.arange(padlen, dtype=jnp.int32)[None, :] % (npad - n),
        (NW, padlen),
    )
    src2 = jnp.concatenate([srcw, pad_src], axis=1).reshape(NW, nwin, K)
    dst2 = jnp.concatenate([dstw, pad_dst], axis=1).reshape(NW, nwin, K)
    zeros_d = jnp.zeros((npad, d), jnp.float32)
    ones_nd = jnp.ones((n, d), jnp.float32)
    b1r = b1.reshape(1, d)
    b2r = b2.reshape(1, d)

    prop = _make_prop_kernel(npad, d, nwin)
    deg16 = prop(ones_nd, src2, dst2, zeros_d)

    h1, g1 = pl.pallas_call(
        _tc1_body,
        grid=grid,
        in_specs=[_deg_spec(), _row_spec(d), _full_spec(d, d)],
        out_specs=[_row_spec(d), _row_spec(d)],
        out_shape=[
            jax.ShapeDtypeStruct((n, d), jnp.float32),
            jax.ShapeDtypeStruct((n, d), jnp.float32),
        ],
    )(deg16, x, W1)

    s1 = prop(g1, src2, dst2, zeros_d)

    h2, g2 = pl.pallas_call(
        _tc2_body,
        grid=grid,
        in_specs=[
            _deg_spec(),
            _pair_spec(d),
            _row_spec(d),
            _full_spec(d, d),
            _full_spec(1, d),
        ],
        out_specs=[_row_spec(d), _row_spec(d)],
        out_shape=[
            jax.ShapeDtypeStruct((n, d), jnp.float32),
            jax.ShapeDtypeStruct((n, d), jnp.float32),
        ],
    )(deg16, s1, h1, W2, b1r)

    s2 = prop(g2, src2, dst2, zeros_d)

    out = pl.pallas_call(
        _tc3_body,
        grid=grid,
        in_specs=[_deg_spec(), _pair_spec(d), _row_spec(d), _full_spec(1, d)],
        out_specs=_row_spec(d),
        out_shape=jax.ShapeDtypeStruct((n, d), jnp.float32),
    )(deg16, s2, h2, b2r)

    return out


# speedup vs baseline: 3.6202x; 1.5187x over previous
import functools

import jax
import jax.numpy as jnp
from jax import lax
from jax.experimental import pallas as pl
from jax.experimental.pallas import tpu as pltpu
from jax.experimental.pallas import tpu_sc as plsc

NC = 2
NS = 16
NW = NC * NS
K = 128
C = 8
BLK = 2000


def _sc_mesh():
    return plsc.VectorSubcoreMesh(
        core_axis_name="c", subcore_axis_name="s", num_cores=NC, num_subcores=NS
    )


def _make_prop_kernel(npad, d, nwin):
    rpt = npad // NS
    nchunk = nwin // C

    @functools.partial(
        pl.kernel,
        out_type=jax.ShapeDtypeStruct((NC, npad, d), jnp.float32),
        mesh=_sc_mesh(),
        scratch_types=[
            pltpu.VMEM((2, C, K), jnp.int32),
            pltpu.VMEM((nwin, K), jnp.int32),
            pltpu.VMEM((2, K, d), jnp.float32),
            pltpu.VMEM_SHARED((npad, d), jnp.float32),
            pltpu.SemaphoreType.DMA,
            pltpu.SemaphoreType.DMA,
            pltpu.SemaphoreType.DMA,
            pltpu.SemaphoreType.DMA,
        ],
    )
    def prop_kernel(
        g_hbm, src_hbm, dst_hbm, zeros_hbm, out_hbm,
        src_v, dst_v, rows_v, acc_sh, gsem0, gsem1, csem0, csem1,
    ):
        gsem = (gsem0, gsem1)
        csem = (csem0, csem1)
        c = lax.axis_index("c")
        s = lax.axis_index("s")
        wid = s * NC + c

        def wait_g(b):
            pltpu.make_async_copy(
                g_hbm.at[pl.ds(0, K)], rows_v.at[b], gsem[b]
            ).wait()

        def wait_c(sl):
            pltpu.make_async_copy(
                src_hbm.at[wid, pl.ds(0, C)], src_v.at[sl], csem[sl]
            ).wait()

        pltpu.sync_copy(dst_hbm.at[wid], dst_v)
        pltpu.sync_copy(src_hbm.at[wid, pl.ds(0, C)], src_v.at[0])
        pltpu.sync_copy(
            zeros_hbm.at[pl.ds(s * rpt, rpt)], acc_sh.at[pl.ds(s * rpt, rpt)]
        )
        plsc.subcore_barrier()
        pltpu.async_copy(g_hbm.at[src_v.at[0, 0]], rows_v.at[0], gsem[0])
        pltpu.async_copy(g_hbm.at[src_v.at[0, 1]], rows_v.at[1], gsem[1])

        @pl.loop(0, nchunk, step=2)
        def _(q0):
            for qq in range(2):
                q = q0 + qq
                sl = qq
                so = 1 - qq

                @pl.when(q + 1 < nchunk)
                def _():
                    pltpu.async_copy(
                        src_hbm.at[wid, pl.ds((q + 1) * C, C)], src_v.at[so],
                        csem[so],
                    )

                for j in range(C):
                    b = j % 2
                    w = q * C + j
                    wait_g(b)
                    pltpu.sync_copy(
                        rows_v.at[b], acc_sh.at[dst_v.at[w]], add=True
                    )
                    if j == C - 2:
                        @pl.when(q + 1 < nchunk)
                        def _():
                            wait_c(so)
                    nsl, nj = (sl, j + 2) if j < C - 2 else (so, j + 2 - C)

                    @pl.when(w + 2 < nwin)
                    def _():
                        pltpu.async_copy(
                            g_hbm.at[src_v.at[nsl, nj]], rows_v.at[b], gsem[b]
                        )

        plsc.subcore_barrier()
        pltpu.sync_copy(
            acc_sh.at[pl.ds(s * rpt, rpt)], out_hbm.at[c, pl.ds(s * rpt, rpt)]
        )

    return prop_kernel


def _dinv_from_deg(deg_ref):
    deg = deg_ref[0, :, 0:1] + deg_ref[1, :, 0:1] + 1.0
    return lax.rsqrt(deg)


def _deg_spec():
    return pl.BlockSpec((2, BLK, 128), lambda i: (0, i, 0))


def _tc1_body(deg_ref, x_ref, w_ref, h_ref, g_ref):
    dinv = _dinv_from_deg(deg_ref)
    h = jnp.dot(x_ref[...], w_ref[...], preferred_element_type=jnp.float32)
    h_ref[...] = h
    g_ref[...] = h * dinv


def _tc2_body(deg_ref, s_ref, h_ref, w_ref, b_ref, h2_ref, g2_ref):
    dinv = _dinv_from_deg(deg_ref)
    stot = s_ref[0] + s_ref[1]
    out1 = jnp.maximum(stot * dinv + h_ref[...] * (dinv * dinv) + b_ref[...], 0.0)
    h2 = jnp.dot(out1, w_ref[...], preferred_element_type=jnp.float32)
    h2_ref[...] = h2
    g2_ref[...] = h2 * dinv


def _tc3_body(deg_ref, s_ref, h_ref, b_ref, o_ref):
    dinv = _dinv_from_deg(deg_ref)
    stot = s_ref[0] + s_ref[1]
    o_ref[...] = jnp.maximum(
        stot * dinv + h_ref[...] * (dinv * dinv) + b_ref[...], 0.0
    )


def _row_spec(d):
    return pl.BlockSpec((BLK, d), lambda i: (i, 0))


def _pair_spec(d):
    return pl.BlockSpec((2, BLK, d), lambda i: (0, i, 0))


def _full_spec(r, c):
    return pl.BlockSpec((r, c), lambda i: (0, 0))


def kernel(x, edge_index, W1, b1, W2, b2):
    n, d = x.shape
    e = edge_index.shape[1]
    grid = (n // BLK,)
    npad = ((n // NS + 7) // 8 * 8) * NS

    epw = e // NW
    nwin_min = (epw + K - 1) // K
    nwin = (nwin_min + 2 * C - 1) // (2 * C) * (2 * C)
    padlen = nwin * K - epw
    srcw = edge_index[0].reshape(NW, epw)
    dstw = edge_index[1].reshape(NW, epw)
    pad_src = jnp.broadcast_to(
        jnp.arange(padlen, dtype=jnp.int32)[None, :] % n, (NW, padlen)
    )
    pad_dst = jnp.broadcast_to(
        n + jnp.arange(padlen, dtype=jnp.int32)[None, :] % (npad - n),
        (NW, padlen),
    )
    src2 = jnp.concatenate([srcw, pad_src], axis=1).reshape(NW, nwin, K)
    dst2 = jnp.concatenate([dstw, pad_dst], axis=1).reshape(NW, nwin, K)
    zeros_d = jnp.zeros((npad, d), jnp.float32)
    ones_nd = jnp.ones((n, d), jnp.float32)
    b1r = b1.reshape(1, d)
    b2r = b2.reshape(1, d)

    prop = _make_prop_kernel(npad, d, nwin)
    deg16 = prop(ones_nd, src2, dst2, zeros_d)

    h1, g1 = pl.pallas_call(
        _tc1_body,
        grid=grid,
        in_specs=[_deg_spec(), _row_spec(d), _full_spec(d, d)],
        out_specs=[_row_spec(d), _row_spec(d)],
        out_shape=[
            jax.ShapeDtypeStruct((n, d), jnp.float32),
            jax.ShapeDtypeStruct((n, d), jnp.float32),
        ],
    )(deg16, x, W1)

    s1 = prop(g1, src2, dst2, zeros_d)

    h2, g2 = pl.pallas_call(
        _tc2_body,
        grid=grid,
        in_specs=[
            _deg_spec(),
            _pair_spec(d),
            _row_spec(d),
            _full_spec(d, d),
            _full_spec(1, d),
        ],
        out_specs=[_row_spec(d), _row_spec(d)],
        out_shape=[
            jax.ShapeDtypeStruct((n, d), jnp.float32),
            jax.ShapeDtypeStruct((n, d), jnp.float32),
        ],
    )(deg16, s1, h1, W2, b1r)

    s2 = prop(g2, src2, dst2, zeros_d)

    out = pl.pallas_call(
        _tc3_body,
        grid=grid,
        in_specs=[_deg_spec(), _pair_spec(d), _row_spec(d), _full_spec(1, d)],
        out_specs=_row_spec(d),
        out_shape=jax.ShapeDtypeStruct((n, d), jnp.float32),
    )(deg16, s2, h2, b2r)

    return out
